# baseline (speedup 1.0000x reference)
import functools

import jax
import jax.numpy as jnp
from jax.experimental import pallas as pl


def _pick_bm(M, cap=2048):
    for c in (2048, 2000, 1600, 1280, 1024, 1000, 800, 640, 512, 400, 320,
              256, 200, 160, 128, 100, 80, 64, 40, 32, 16, 8):
        if c <= cap and M % c == 0:
            return c
    return M


def _mm_kernel(x_ref, w_ref, b_ref, o_ref, *, act):
    y = jnp.dot(x_ref[...], w_ref[...], preferred_element_type=jnp.float32)
    y = y + b_ref[...]
    if act:
        y = jnp.maximum(y, 0.0)
    o_ref[...] = y


def _matmul_bias(x, w, b, act, bm=None):
    M, K = x.shape
    Hh = w.shape[1]
    bm = bm or _pick_bm(M)
    return pl.pallas_call(
        functools.partial(_mm_kernel, act=act),
        grid=(M // bm,),
        in_specs=[
            pl.BlockSpec((bm, K), lambda i: (i, 0)),
            pl.BlockSpec((K, Hh), lambda i: (0, 0)),
            pl.BlockSpec((1, Hh), lambda i: (0, 0)),
        ],
        out_specs=pl.BlockSpec((bm, Hh), lambda i: (i, 0)),
        out_shape=jax.ShapeDtypeStruct((M, Hh), jnp.float32),
    )(x, w, b.reshape(1, Hh))


def _edge_update_kernel(m_ref, w_ref, b_ref, h0_ref, o_ref):
    y = jnp.dot(m_ref[...], w_ref[...], preferred_element_type=jnp.float32)
    y = y + b_ref[...] + h0_ref[...]
    o_ref[...] = jnp.maximum(y, 0.0)


def _edge_update(m, w, b, h0e, bm=None):
    M, K = m.shape
    Hh = w.shape[1]
    bm = bm or _pick_bm(M)
    return pl.pallas_call(
        _edge_update_kernel,
        grid=(M // bm,),
        in_specs=[
            pl.BlockSpec((bm, K), lambda i: (i, 0)),
            pl.BlockSpec((K, Hh), lambda i: (0, 0)),
            pl.BlockSpec((1, Hh), lambda i: (0, 0)),
            pl.BlockSpec((bm, Hh), lambda i: (i, 0)),
        ],
        out_specs=pl.BlockSpec((bm, Hh), lambda i: (i, 0)),
        out_shape=jax.ShapeDtypeStruct((M, Hh), jnp.float32),
    )(m, w, b.reshape(1, Hh), h0e)


def _head_kernel(mkn_ref, hkn_ref, h0n_ref, w1_ref, w2_ref, w3_ref, bl_ref,
                 wo_ref, bo_ref, o_ref, *, nblocks, n_rows):
    i = pl.program_id(0)
    h = jnp.dot(mkn_ref[...], w1_ref[...], preferred_element_type=jnp.float32)
    h += jnp.dot(hkn_ref[...], w2_ref[...], preferred_element_type=jnp.float32)
    h += jnp.dot(h0n_ref[...], w3_ref[...], preferred_element_type=jnp.float32)
    h += bl_ref[...]
    h2 = jnp.dot(h, wo_ref[...], preferred_element_type=jnp.float32) + bo_ref[...]
    h2 = jnp.maximum(h2, 0.0)
    part = jnp.sum(h2, axis=0, keepdims=True) * (1.0 / n_rows)

    @pl.when(i == 0)
    def _():
        o_ref[...] = jnp.zeros_like(o_ref)

    o_ref[...] += part


def _head(mkn, hkn, h0n, W_lin, b_lin, W_o, b_o, bm=None):
    M, Hh = hkn.shape
    bm = bm or _pick_bm(M)
    w1 = W_lin[:Hh]
    w2 = W_lin[Hh:2 * Hh]
    w3 = W_lin[2 * Hh:]
    nblocks = M // bm
    row_spec = pl.BlockSpec((bm, Hh), lambda i: (i, 0))
    w_spec = pl.BlockSpec((Hh, Hh), lambda i: (0, 0))
    b_spec = pl.BlockSpec((1, Hh), lambda i: (0, 0))
    return pl.pallas_call(
        functools.partial(_head_kernel, nblocks=nblocks, n_rows=M),
        grid=(nblocks,),
        in_specs=[row_spec, row_spec, row_spec, w_spec, w_spec, w_spec,
                  b_spec, w_spec, b_spec],
        out_specs=pl.BlockSpec((1, Hh), lambda i: (0, 0)),
        out_shape=jax.ShapeDtypeStruct((1, Hh), jnp.float32),
    )(mkn, hkn, h0n, w1, w2, w3, b_lin.reshape(1, Hh), W_o,
      b_o.reshape(1, Hh))


def kernel(node_attr, edge_attr, edge_index, W_atom, b_atom, W_bond, b_bond,
           W_h, b_h, W_o, b_o, W_lin, b_lin):
    N = node_attr.shape[0]
    E = edge_attr.shape[0]
    depth = W_h.shape[0] + 1
    dst = edge_index[1]

    h0n = _matmul_bias(node_attr, W_atom, b_atom, act=True)
    h0e = _matmul_bias(edge_attr, W_bond, b_bond, act=True)
    hkn, hke = h0n, h0e

    for k in range(depth - 1):
        hs = jax.ops.segment_sum(hke, dst, num_segments=N)
        hm = jax.ops.segment_max(hke, dst, num_segments=N)
        hm = jnp.where(jnp.isfinite(hm), hm, 0.0)
        hkn = hkn + hs * hm
        G = jnp.take(hkn, dst, axis=0)
        Rv = hke.reshape(E // 2, 2, -1)[:, ::-1, :].reshape(E, -1)
        m = G - Rv
        hke = _edge_update(m, W_h[k], b_h[k], h0e)

    hs = jax.ops.segment_sum(hke, dst, num_segments=N)
    hm = jax.ops.segment_max(hke, dst, num_segments=N)
    hm = jnp.where(jnp.isfinite(hm), hm, 0.0)
    mkn = hs * hm
    return _head(mkn, hkn, h0n, W_lin, b_lin, W_o, b_o)


# speedup vs baseline: 1.2474x; 1.2474x over previous
import functools

import jax
import jax.numpy as jnp
from jax import lax
from jax.experimental import pallas as pl
from jax.experimental.pallas import tpu as pltpu
from jax.experimental.pallas import tpu_sc as plsc

_NC = 2
_NS = 16
_NW = _NC * _NS


def _pick_bm(M, cap=2048):
    for c in (2048, 2000, 1600, 1280, 1024, 1000, 800, 640, 512, 400, 320,
              256, 200, 160, 128, 100, 80, 64, 40, 32, 16, 8):
        if c <= cap and M % c == 0:
            return c
    return M


def _mm_kernel(x_ref, w_ref, b_ref, o_ref, *, act):
    y = jnp.dot(x_ref[...], w_ref[...], preferred_element_type=jnp.float32)
    y = y + b_ref[...]
    if act:
        y = jnp.maximum(y, 0.0)
    o_ref[...] = y


def _matmul_bias(x, w, b, act, bm=None):
    M, K = x.shape
    Hh = w.shape[1]
    bm = bm or _pick_bm(M)
    return pl.pallas_call(
        functools.partial(_mm_kernel, act=act),
        grid=(M // bm,),
        in_specs=[
            pl.BlockSpec((bm, K), lambda i: (i, 0)),
            pl.BlockSpec((K, Hh), lambda i: (0, 0)),
            pl.BlockSpec((1, Hh), lambda i: (0, 0)),
        ],
        out_specs=pl.BlockSpec((bm, Hh), lambda i: (i, 0)),
        out_shape=jax.ShapeDtypeStruct((M, Hh), jnp.float32),
    )(x, w, b.reshape(1, Hh))


def _edge_update_kernel(g_ref, r_ref, w_ref, b_ref, h0_ref, o_ref):
    m = g_ref[...] - r_ref[...]
    y = jnp.dot(m, w_ref[...], preferred_element_type=jnp.float32)
    y = y + b_ref[...] + h0_ref[...]
    o_ref[...] = jnp.maximum(y, 0.0)


def _edge_update(g, r, w, b, h0e, bm=None):
    M, K = g.shape
    Hh = w.shape[1]
    bm = bm or _pick_bm(M)
    row = pl.BlockSpec((bm, K), lambda i: (i, 0))
    return pl.pallas_call(
        _edge_update_kernel,
        grid=(M // bm,),
        in_specs=[
            row, row,
            pl.BlockSpec((K, Hh), lambda i: (0, 0)),
            pl.BlockSpec((1, Hh), lambda i: (0, 0)),
            pl.BlockSpec((bm, Hh), lambda i: (i, 0)),
        ],
        out_specs=pl.BlockSpec((bm, Hh), lambda i: (i, 0)),
        out_shape=jax.ShapeDtypeStruct((M, Hh), jnp.float32),
    )(g, r, w, b.reshape(1, Hh), h0e)


def _combine_kernel(hs_ref, hm_ref, hkn_ref, hkn_o, mkn_o):
    m = hs_ref[...] * hm_ref[...]
    mkn_o[...] = m
    hkn_o[...] = hkn_ref[...] + m


def _combine(hs, hm, hkn, bm=None):
    M, Hh = hs.shape
    bm = bm or _pick_bm(M)
    row = pl.BlockSpec((bm, Hh), lambda i: (i, 0))
    return pl.pallas_call(
        _combine_kernel,
        grid=(M // bm,),
        in_specs=[row, row, row],
        out_specs=[row, row],
        out_shape=[jax.ShapeDtypeStruct((M, Hh), jnp.float32),
                   jax.ShapeDtypeStruct((M, Hh), jnp.float32)],
    )(hs, hm, hkn)


def _head_kernel(mkn_ref, hkn_ref, h0n_ref, w1_ref, w2_ref, w3_ref, bl_ref,
                 wo_ref, bo_ref, o_ref, *, n_rows):
    i = pl.program_id(0)
    h = jnp.dot(mkn_ref[...], w1_ref[...], preferred_element_type=jnp.float32)
    h += jnp.dot(hkn_ref[...], w2_ref[...], preferred_element_type=jnp.float32)
    h += jnp.dot(h0n_ref[...], w3_ref[...], preferred_element_type=jnp.float32)
    h += bl_ref[...]
    h2 = jnp.dot(h, wo_ref[...], preferred_element_type=jnp.float32) + bo_ref[...]
    h2 = jnp.maximum(h2, 0.0)
    part = jnp.sum(h2, axis=0, keepdims=True) * (1.0 / n_rows)

    @pl.when(i == 0)
    def _():
        o_ref[...] = jnp.zeros_like(o_ref)

    o_ref[...] += part


def _head(mkn, hkn, h0n, W_lin, b_lin, W_o, b_o, bm=None):
    M, Hh = hkn.shape
    bm = bm or _pick_bm(M)
    row_spec = pl.BlockSpec((bm, Hh), lambda i: (i, 0))
    w_spec = pl.BlockSpec((Hh, Hh), lambda i: (0, 0))
    b_spec = pl.BlockSpec((1, Hh), lambda i: (0, 0))
    return pl.pallas_call(
        functools.partial(_head_kernel, n_rows=M),
        grid=(M // bm,),
        in_specs=[row_spec, row_spec, row_spec, w_spec, w_spec, w_spec,
                  b_spec, w_spec, b_spec],
        out_specs=pl.BlockSpec((1, Hh), lambda i: (0, 0)),
        out_shape=jax.ShapeDtypeStruct((1, Hh), jnp.float32),
    )(mkn, hkn, h0n, W_lin[:Hh], W_lin[Hh:2 * Hh], W_lin[2 * Hh:],
      b_lin.reshape(1, Hh), W_o, b_o.reshape(1, Hh))


_SEG_B = 48
_SEG_SB = 32


@functools.lru_cache(maxsize=None)
def _build_segreduce(E, N, H):
    NCH = H // 16
    B, SB = _SEG_B, _SEG_SB
    mesh = plsc.VectorSubcoreMesh(core_axis_name="c", subcore_axis_name="s")
    NEG = jnp.float32(-3.0e38)

    @functools.partial(
        pl.kernel, mesh=mesh,
        out_type=(jax.ShapeDtypeStruct((N, H), jnp.float32),
                  jax.ShapeDtypeStruct((N, H), jnp.float32)),
        scratch_types=[
            pltpu.VMEM((B + 8, H), jnp.float32),
            pltpu.VMEM((B + 40,), jnp.int32),
            pltpu.VMEM((96,), jnp.int32),
            pltpu.VMEM((SB, H), jnp.float32),
            pltpu.VMEM((SB, H), jnp.float32),
            pltpu.VMEM((H,), jnp.float32),
            pltpu.VMEM((H,), jnp.float32),
            pltpu.SMEM((8,), jnp.int32),
        ],
    )
    def seg(hke_hbm, dst_hbm, meta_hbm, hs_hbm, hm_hbm,
            ibuf, dbuf, meta_v, stage_s, stage_m, accs, accm, state):
        wid = lax.axis_index("s") * _NC + lax.axis_index("c")
        pltpu.sync_copy(meta_hbm, meta_v)
        def mread(i):
            return meta_v[pl.ds(i, 16)][0]
        s0 = mread(wid)
        s1 = mread(wid + 1)
        zlo = mread(33 + wid)
        zhi = mread(33 + wid + 1)

        state[0] = jnp.int32(-1)
        state[1] = zlo
        state[2] = jnp.int32(0)
        state[3] = zlo

        zvec = jnp.zeros((16,), jnp.float32)

        def bump_stage():
            scnt = state[2] + 1
            sbase = state[3]
            full = scnt == SB

            @pl.when(full)
            def _dma():
                sba = pl.multiple_of(sbase, 8)
                pltpu.sync_copy(stage_s, hs_hbm.at[pl.ds(sba, SB)])
                pltpu.sync_copy(stage_m, hm_hbm.at[pl.ds(sba, SB)])

            state[2] = jnp.where(full, jnp.int32(0), scnt)
            state[3] = jnp.where(full, sbase + SB, sbase)

        def push_zero(i, zcarry):
            scnt = state[2]
            for c in range(NCH):
                stage_s[scnt, pl.ds(c * 16, 16)] = zvec
                stage_m[scnt, pl.ds(c * 16, 16)] = zvec
            bump_stage()
            return zcarry

        def flush_segment():
            prev_d = state[0]
            lax.fori_loop(state[1], prev_d, push_zero, None)
            scnt = state[2]
            for c in range(NCH):
                sl = pl.ds(c * 16, 16)
                stage_s[scnt, sl] = accs[sl]
                stage_m[scnt, sl] = accm[sl]
            bump_stage()
            state[1] = prev_d + 1

        cnt = s1 - s0
        nb = (cnt + B - 1) // B

        def batch_body(bi, carry):
            e0 = s0 + bi * B
            er = e0 - (e0 % 8)
            e0f = pl.multiple_of(jnp.minimum(er, E - (B + 8)), 8)
            pltpu.sync_copy(hke_hbm.at[pl.ds(e0f, B + 8)], ibuf)
            e0a = pl.multiple_of(er, 8)
            pltpu.sync_copy(dst_hbm.at[pl.ds(e0a, B + 40)], dbuf)

            def row_body(r, rcarry):
                e = e0 + r

                @pl.when(e < s1)
                def _process():
                    d = dbuf[pl.ds(e - e0a, 16)][0]
                    prev_d = state[0]
                    keep = d == prev_d

                    @pl.when(jnp.logical_and(jnp.logical_not(keep),
                                             prev_d >= 0))
                    def _flush():
                        flush_segment()

                    ri = e - e0f
                    for c in range(NCH):
                        sl = pl.ds(c * 16, 16)
                        v = ibuf[ri, sl]
                        accs[sl] = jnp.where(keep, accs[sl] + v, v)
                        accm[sl] = jnp.maximum(v, jnp.where(keep, accm[sl], NEG))
                    state[0] = d
                return rcarry

            lax.fori_loop(0, B, row_body, None)
            return carry

        lax.fori_loop(0, nb, batch_body, None)

        @pl.when(state[0] >= 0)
        def _final():
            flush_segment()

        lax.fori_loop(state[1], zhi, push_zero, None)
        scnt = state[2]
        sbase = state[3]

        def tail_body(j, tcarry):
            dsta = pl.multiple_of(sbase + 8 * j, 8)
            pltpu.sync_copy(stage_s.at[pl.ds(8 * j, 8)],
                            hs_hbm.at[pl.ds(dsta, 8)])
            pltpu.sync_copy(stage_m.at[pl.ds(8 * j, 8)],
                            hm_hbm.at[pl.ds(dsta, 8)])
            return tcarry

        lax.fori_loop(0, scnt // 8, tail_body, None)

    return seg


def _segreduce(hke_s, dst_s_pad, meta, N):
    E, H = hke_s.shape
    return _build_segreduce(E, N, H)(hke_s, dst_s_pad, meta)


def kernel(node_attr, edge_attr, edge_index, W_atom, b_atom, W_bond, b_bond,
           W_h, b_h, W_o, b_o, W_lin, b_lin):
    N = node_attr.shape[0]
    E = edge_attr.shape[0]
    depth = W_h.shape[0] + 1
    dst = edge_index[1]

    order = jnp.argsort(dst)
    dst_s = jnp.take(dst, order)
    inv = jnp.zeros((E,), jnp.int32).at[order].set(
        jnp.arange(E, dtype=jnp.int32))
    eidx = jnp.arange(E, dtype=jnp.int32)
    rev = eidx + 1 - 2 * (eidx % 2)
    perm_rev = jnp.take(inv, jnp.take(rev, order))
    t = (jnp.arange(_NW + 1, dtype=jnp.int32) * E) // _NW
    nw_node = jnp.take(dst_s, jnp.minimum(t, E - 1))
    zb = (nw_node // 8) * 8
    zb = zb.at[0].set(0).at[_NW].set(N)
    starts = jnp.searchsorted(dst_s, zb, side="left").astype(jnp.int32)
    meta = jnp.zeros((96,), jnp.int32)
    meta = meta.at[0:_NW + 1].set(starts)
    meta = meta.at[33:33 + _NW + 1].set(zb)
    dst_s_pad = jnp.concatenate(
        [dst_s, jnp.zeros((_SEG_B + 40,), jnp.int32)])

    h0n = _matmul_bias(node_attr, W_atom, b_atom, act=True)
    h0e = _matmul_bias(jnp.take(edge_attr, order, axis=0), W_bond, b_bond,
                       act=True)
    hkn, hke = h0n, h0e

    for k in range(depth - 1):
        hs, hm = _segreduce(hke, dst_s_pad, meta, N)
        hkn, _ = _combine(hs, hm, hkn)
        g = jnp.take(hkn, dst_s, axis=0)
        r = jnp.take(hke, perm_rev, axis=0)
        hke = _edge_update(g, r, W_h[k], b_h[k], h0e)

    hs, hm = _segreduce(hke, dst_s_pad, meta, N)
    _, mkn = _combine(hs, hm, hkn)
    return _head(mkn, hkn, h0n, W_lin, b_lin, W_o, b_o)
